# Initial kernel scaffold; baseline (speedup 1.0000x reference)
import jax
import jax.numpy as jnp
from jax.experimental import pallas as pl


def kernel(x):
    raise NotImplementedError("write your pallas kernel here")


# speedup vs baseline: 1.4457x; 1.4457x over previous
import jax
import jax.numpy as jnp
from jax.experimental import pallas as pl
from jax.experimental.pallas import tpu as pltpu

_R = 512


def _body(xl_ref, x_ref, o_ref, prev_ref):
    i = pl.program_id(1)
    cur = x_ref[0]
    first = jnp.where(i == 0, xl_ref[0], prev_ref[...])
    shifted = jnp.concatenate([first, cur[:-1]], axis=0)
    o_ref[0] = cur - shifted
    prev_ref[...] = cur[_R - 1:_R]


def kernel(x):
    B, S, C = x.shape
    nb = S // _R
    xlast = x[:, S - 1:, :]
    return pl.pallas_call(
        _body,
        grid=(B, nb),
        in_specs=[
            pl.BlockSpec((1, 1, C), lambda b, i: (b, 0, 0)),
            pl.BlockSpec((1, _R, C), lambda b, i: (b, i, 0)),
        ],
        out_specs=pl.BlockSpec((1, _R, C), lambda b, i: (b, i, 0)),
        out_shape=jax.ShapeDtypeStruct((B, S, C), x.dtype),
        scratch_shapes=[pltpu.VMEM((1, C), x.dtype)],
    )(xlast, x)
